# Initial kernel scaffold; baseline (speedup 1.0000x reference)
import jax
import jax.numpy as jnp
from jax.experimental import pallas as pl


def kernel(x_User, x_Resource, node_id_User, node_id_Resource, node_id_Title, node_id_Manager, node_id_Department, ei_access, ei_rev, ei_title, ei_mgr, ei_dept, edge_label_index, params):
    raise NotImplementedError("write your pallas kernel here")


# speedup vs baseline: 1.3193x; 1.3193x over previous
import functools

import jax
import jax.numpy as jnp
from jax import lax
from jax.experimental import pallas as pl
from jax.experimental.pallas import tpu as pltpu
from jax.experimental.pallas import tpu_sc as plsc

NC, NS, L = 2, 16, 16
H = 128
F32 = jnp.float32


def _cdiv(a, b):
    return (a + b - 1) // b


def _mesh():
    return plsc.VectorSubcoreMesh(core_axis_name="c", subcore_axis_name="s")


def _counts_call(dst_list, nd_list):
    n_t = len(dst_list)
    e_list = [int(e.shape[0]) for e in dst_list]
    CH = 512
    NB = CH // 128
    ndp_list = [nd + L for nd in nd_list]
    ZR = 512

    scratch = [
        pltpu.VMEM((CH,), jnp.int32),
        pltpu.VMEM((NB, 128), jnp.int32),
        pltpu.VMEM((128, L), F32),
        pltpu.VMEM((ZR, L), F32),
        pltpu.VMEM_SHARED((max(ndp_list), L), F32),
        pltpu.SemaphoreType.DMA,
    ]
    out_type = tuple(jax.ShapeDtypeStruct((NC, nd, L), F32) for nd in nd_list)

    @functools.partial(pl.kernel, out_type=out_type, mesh=_mesh(),
                       scratch_types=scratch,
                       compiler_params=pltpu.CompilerParams(
                           use_tc_tiling_on_sc=False,
                           needs_layout_passes=False))
    def kern(*refs):
        dsts = refs[:n_t]
        outs = refs[n_t:2 * n_t]
        stage, didx, onesbuf, zbuf, acc, sem = refs[2 * n_t:]
        c = lax.axis_index("c")
        s = lax.axis_index("s")
        w = s * NC + c
        lanes = lax.iota(jnp.int32, L)

        def ib(i, _):
            onesbuf[i, pl.ds(0, L)] = jnp.ones((L,), F32)
            return 0
        lax.fori_loop(0, 128, ib, 0)

        def zb(i, _):
            zbuf[i, pl.ds(0, L)] = jnp.zeros((L,), F32)
            return 0
        lax.fori_loop(0, ZR, zb, 0)

        for t in range(n_t):
            E, ND, NDP = e_list[t], nd_list[t], ndp_list[t]
            sp = NDP // NS
            row_base = s * sp
            for k in range(_cdiv(sp, ZR)):
                r0 = jnp.minimum(row_base + k * ZR, NDP - ZR)
                pltpu.sync_copy(zbuf, acc.at[pl.ds(r0, ZR), :])
            plsc.subcore_barrier()

            nch = _cdiv(E, CH)

            def chunk(i, _):
                g = w + i * NC * NS

                @pl.when(g < nch)
                def _():
                    start = g * CH
                    off = jnp.minimum(start, E - CH)
                    pltpu.sync_copy(dsts[t].at[pl.ds(off, CH)], stage)
                    for j in range(CH // L):
                        dv = stage[pl.ds(j * L, L)]
                        pos = off + j * L + lanes
                        valid = (pos >= start) & (pos < E)
                        b, r = j // (128 // L), j % (128 // L)
                        didx[b, pl.ds(r * L, L)] = jnp.where(valid, dv, ND)
                    scs = [pltpu.async_copy(onesbuf, acc.at[didx.at[b]],
                                            sem, add=True)
                           for b in range(NB)]
                    for cp in scs:
                        cp.wait()
                return 0
            lax.fori_loop(0, _cdiv(nch, NC * NS), chunk, 0)
            plsc.subcore_barrier()

            @pl.when(s < NS - 1)
            def _():
                pltpu.sync_copy(acc.at[pl.ds(row_base, sp), :],
                                outs[t].at[c, pl.ds(row_base, sp), :])

            @pl.when(s == NS - 1)
            def _():
                pltpu.sync_copy(acc.at[pl.ds(row_base, sp - L), :],
                                outs[t].at[c, pl.ds(row_base, sp - L), :])
            plsc.subcore_barrier()

    return kern(*dst_list)


def _agg_call(y_flat, e_src, e_dst, nd, seg_total):
    E = int(e_src.shape[0])
    G = H // seg_total
    P = seg_total // NC
    NDP = nd + L
    sp = NDP // NS
    assert sp * NS == NDP
    CE = 512 if G <= 16 else 256
    NB = CE // 128
    ZR = 8192 // G

    scratch = [
        pltpu.VMEM((CE,), jnp.int32),
        pltpu.VMEM((CE,), jnp.int32),
        pltpu.VMEM((NB, 128), jnp.int32),
        pltpu.VMEM((NB, 128), jnp.int32),
        pltpu.VMEM((CE, G), F32),
        pltpu.VMEM((ZR, G), F32),
        pltpu.VMEM_SHARED((NDP, G), F32),
        pltpu.SemaphoreType.DMA,
        pltpu.SemaphoreType.DMA,
    ]
    out_type = jax.ShapeDtypeStruct((nd, seg_total, G), F32)

    @functools.partial(pl.kernel, out_type=out_type, mesh=_mesh(),
                       scratch_types=scratch,
                       compiler_params=pltpu.CompilerParams(
                           use_tc_tiling_on_sc=False,
                           needs_layout_passes=False))
    def kern(yf, esr, edr, out, ssrc, sdst, gidx, didx, rowbuf, zbuf, acc,
             gsem, ssem):
        c = lax.axis_index("c")
        s = lax.axis_index("s")
        lanes = lax.iota(jnp.int32, L)

        def zz(i, _):
            for kk in range(G // L):
                zbuf[i, pl.ds(kk * L, L)] = jnp.zeros((L,), F32)
            return 0
        lax.fori_loop(0, ZR, zz, 0)

        nch = _cdiv(E, CE)
        nzc = _cdiv(sp, ZR)

        def one_pass(p, _):
            segidx = c * P + p
            row_base = s * sp
            for k in range(nzc):
                r0 = jnp.minimum(row_base + k * ZR, NDP - ZR)
                pltpu.sync_copy(zbuf, acc.at[pl.ds(r0, ZR), :])
            plsc.subcore_barrier()

            def chunk(i, _):
                g = s + i * NS

                @pl.when(g < nch)
                def _():
                    start = g * CE
                    off = jnp.minimum(start, E - CE)
                    pltpu.sync_copy(esr.at[pl.ds(off, CE)], ssrc)
                    pltpu.sync_copy(edr.at[pl.ds(off, CE)], sdst)
                    for j in range(CE // L):
                        sv = ssrc[pl.ds(j * L, L)]
                        dv = sdst[pl.ds(j * L, L)]
                        pos = off + j * L + lanes
                        valid = (pos >= start) & (pos < E)
                        gi = jnp.where(valid, sv * seg_total + segidx, 0)
                        di = jnp.where(valid, dv, nd)
                        b, r = j // (128 // L), j % (128 // L)
                        gidx[b, pl.ds(r * L, L)] = gi
                        didx[b, pl.ds(r * L, L)] = di
                    cps = [pltpu.async_copy(yf.at[gidx.at[b]],
                                            rowbuf.at[pl.ds(b * 128, 128), :],
                                            gsem)
                           for b in range(NB)]
                    for cp in cps:
                        cp.wait()
                    scs = [pltpu.async_copy(rowbuf.at[pl.ds(b * 128, 128), :],
                                            acc.at[didx.at[b]],
                                            ssem, add=True)
                           for b in range(NB)]
                    for cp in scs:
                        cp.wait()
                return 0
            lax.fori_loop(0, _cdiv(nch, NS), chunk, 0)
            plsc.subcore_barrier()

            @pl.when(s < NS - 1)
            def _():
                pltpu.sync_copy(acc.at[pl.ds(row_base, sp), :],
                                out.at[pl.ds(row_base, sp), segidx, :])

            @pl.when(s == NS - 1)
            def _():
                pltpu.sync_copy(acc.at[pl.ds(row_base, sp - L), :],
                                out.at[pl.ds(row_base, sp - L), segidx, :])
            plsc.subcore_barrier()
            return 0
        lax.fori_loop(0, P, one_pass, 0)

    return kern(y_flat, e_src, e_dst).reshape(nd, H)


def _pred_call(u, r, el0, el1):
    EL = int(el0.shape[0])
    CP = 128
    scratch = [
        pltpu.VMEM((CP,), jnp.int32),
        pltpu.VMEM((CP,), jnp.int32),
        pltpu.VMEM((CP, H), F32),
        pltpu.VMEM((CP, H), F32),
        pltpu.VMEM((CP,), F32),
        pltpu.SemaphoreType.DMA,
    ]
    out_type = jax.ShapeDtypeStruct((EL,), F32)

    @functools.partial(pl.kernel, out_type=out_type, mesh=_mesh(),
                       scratch_types=scratch,
                       compiler_params=pltpu.CompilerParams(
                           use_tc_tiling_on_sc=False,
                           needs_layout_passes=False))
    def kern(ur, rr, e0r, e1r, out, se0, se1, ubuf, rbuf, pbuf, sem):
        c = lax.axis_index("c")
        s = lax.axis_index("s")
        w = s * NC + c
        nch = _cdiv(EL, CP)

        def chunk(i, _):
            g = w + i * NC * NS

            @pl.when(g < nch)
            def _():
                off = jnp.minimum(g * CP, EL - CP)
                pltpu.sync_copy(e0r.at[pl.ds(off, CP)], se0)
                pltpu.sync_copy(e1r.at[pl.ds(off, CP)], se1)
                cu = pltpu.async_copy(ur.at[se0], ubuf, sem)
                cr = pltpu.async_copy(rr.at[se1], rbuf, sem)
                cu.wait()
                cr.wait()

                lanes = lax.iota(jnp.int32, L)

                def grp(jj, _):
                    base = jj * L
                    res = jnp.zeros((L,), F32)
                    for rr in range(L):
                        j = base + rr
                        acc = ubuf[j, pl.ds(0, L)] * rbuf[j, pl.ds(0, L)]
                        for k in range(1, H // L):
                            acc = acc + (ubuf[j, pl.ds(k * L, L)]
                                         * rbuf[j, pl.ds(k * L, L)])
                        res = jnp.where(lanes == rr, jnp.sum(acc), res)
                    pbuf[pl.ds(base, L)] = res
                    return 0
                lax.fori_loop(0, CP // L, grp, 0)
                pltpu.sync_copy(pbuf, out.at[pl.ds(off, CP)])
            return 0
        lax.fori_loop(0, _cdiv(nch, NC * NS), chunk, 0)

    return kern(u, r, el0, el1)


def _dot(a, b):
    return jnp.dot(a, b, preferred_element_type=F32)


def _row_spec(bm, k):
    return pl.BlockSpec((bm, k), lambda i: (i, 0))


def _w_spec(k):
    return pl.BlockSpec((k, H), lambda i: (0, 0))


def _prep_user(x_u, emb, at_, am_, ad_, it_, im_, id_, wu, bu, wr1, b1,
               w1t, w1m, w1d, w2t, w2m, w2d, b2):
    n = x_u.shape[0]
    BM = 1000

    def body(xu, em, at, am, ad, it, im, idp, wur, bur, wr1r, b1r,
             w1tr, w1mr, w1dr, w2tr, w2mr, w2dr, b2r, oxu0, opre1, opre2):
        x0 = _dot(xu[...], wur[...]) + bur[...] + em[...]
        oxu0[...] = x0
        mt = at[...] * it[...]
        mm = am[...] * im[...]
        md = ad[...] * idp[...]
        opre1[...] = (_dot(x0, wr1r[...]) + b1r[...] + _dot(mt, w1tr[...])
                      + _dot(mm, w1mr[...]) + _dot(md, w1dr[...]))
        opre2[...] = (b2r[...] + _dot(mt, w2tr[...]) + _dot(mm, w2mr[...])
                      + _dot(md, w2dr[...]))

    kf = x_u.shape[1]
    return pl.pallas_call(
        body,
        grid=(n // BM,),
        in_specs=[_row_spec(BM, kf), _row_spec(BM, H), _row_spec(BM, H),
                  _row_spec(BM, H), _row_spec(BM, H), _row_spec(BM, 1),
                  _row_spec(BM, 1), _row_spec(BM, 1), _w_spec(kf),
                  _w_spec(1), _w_spec(H), _w_spec(1), _w_spec(H), _w_spec(H),
                  _w_spec(H), _w_spec(H), _w_spec(H), _w_spec(H), _w_spec(1)],
        out_specs=[_row_spec(BM, H)] * 3,
        out_shape=[jax.ShapeDtypeStruct((n, H), F32)] * 3,
    )(x_u, emb, at_, am_, ad_, it_, im_, id_, wu, bu, wr1, b1,
      w1t, w1m, w1d, w2t, w2m, w2d, b2)


def _prep_res(x_r, emb, wr, br, w1rev):
    n = x_r.shape[0]
    BM = 1000

    def body(xr, em, wrr, brr, w1r, oxr0, oy1):
        x0 = _dot(xr[...], wrr[...]) + brr[...] + em[...]
        oxr0[...] = x0
        oy1[...] = _dot(x0, w1r[...])

    kf = x_r.shape[1]
    return pl.pallas_call(
        body,
        grid=(n // BM,),
        in_specs=[_row_spec(BM, kf), _row_spec(BM, H), _w_spec(kf),
                  _w_spec(1), _w_spec(H)],
        out_specs=[_row_spec(BM, H)] * 2,
        out_shape=[jax.ShapeDtypeStruct((n, H), F32)] * 2,
    )(x_r, emb, wr, br, w1rev)


def _combine_u1(pre1, agg, inv, pres2, wr2s):
    n = pre1.shape[0]
    BM = 1000

    def body(p1, ag, iv, p2, w2, ou1, opre2):
        u = jnp.maximum(p1[...] + iv[...] * ag[...], 0.0)
        ou1[...] = u
        opre2[...] = p2[...] + _dot(u, w2[...])

    return pl.pallas_call(
        body,
        grid=(n // BM,),
        in_specs=[_row_spec(BM, H), _row_spec(BM, H), _row_spec(BM, 1),
                  _row_spec(BM, H), _w_spec(H)],
        out_specs=[_row_spec(BM, H)] * 2,
        out_shape=[jax.ShapeDtypeStruct((n, H), F32)] * 2,
    )(pre1, agg, inv, pres2, wr2s)


def _combine_r1(agg, inv, xr0, wl, wrr, b, w2rev):
    n = agg.shape[0]
    BM = 1000

    def body(ag, iv, x0, wlr, wrr_, br, w2r, or1, oy2):
        rr = jnp.maximum(_dot(ag[...] * iv[...], wlr[...])
                         + _dot(x0[...], wrr_[...]) + br[...], 0.0)
        or1[...] = rr
        oy2[...] = _dot(rr, w2r[...])

    return pl.pallas_call(
        body,
        grid=(n // BM,),
        in_specs=[_row_spec(BM, H), _row_spec(BM, 1), _row_spec(BM, H),
                  _w_spec(H), _w_spec(H), _w_spec(1), _w_spec(H)],
        out_specs=[_row_spec(BM, H)] * 2,
        out_shape=[jax.ShapeDtypeStruct((n, H), F32)] * 2,
    )(agg, inv, xr0, wl, wrr, b, w2rev)


def _combine_u2(pre2, agg, inv):
    n = pre2.shape[0]
    BM = 1000

    def body(p2, ag, iv, ou2):
        ou2[...] = p2[...] + iv[...] * ag[...]

    return pl.pallas_call(
        body,
        grid=(n // BM,),
        in_specs=[_row_spec(BM, H), _row_spec(BM, H), _row_spec(BM, 1)],
        out_specs=_row_spec(BM, H),
        out_shape=jax.ShapeDtypeStruct((n, H), F32),
    )(pre2, agg, inv)


def _combine_r2(agg, inv, r1, wl, wrr, b):
    n = agg.shape[0]
    BM = 1000

    def body(ag, iv, r1r, wlr, wrr_, br, or2):
        or2[...] = (_dot(ag[...] * iv[...], wlr[...])
                    + _dot(r1r[...], wrr_[...]) + br[...])

    return pl.pallas_call(
        body,
        grid=(n // BM,),
        in_specs=[_row_spec(BM, H), _row_spec(BM, 1), _row_spec(BM, H),
                  _w_spec(H), _w_spec(H), _w_spec(1)],
        out_specs=_row_spec(BM, H),
        out_shape=jax.ShapeDtypeStruct((n, H), F32),
    )(agg, inv, r1, wl, wrr, b)


def kernel(x_User, x_Resource, node_id_User, node_id_Resource, node_id_Title,
           node_id_Manager, node_id_Department, ei_access, ei_rev, ei_title,
           ei_mgr, ei_dept, edge_label_index, params):
    p = params
    nu = x_User.shape[0]
    nr = x_Resource.shape[0]
    nt = p['Title_emb'].shape[0]
    nm = p['Manager_emb'].shape[0]
    nd_ = p['Department_emb'].shape[0]

    def col(v):
        return v.reshape(-1, 1)

    def rowv(v):
        return v.reshape(1, -1)

    rev_s, rev_d = ei_rev[0], ei_rev[1]
    acc_s, acc_d = ei_access[0], ei_access[1]
    tit_s, tit_d = ei_title[0], ei_title[1]
    mgr_s, mgr_d = ei_mgr[0], ei_mgr[1]
    dep_s, dep_d = ei_dept[0], ei_dept[1]
    el0, el1 = edge_label_index[0], edge_label_index[1]

    cnt_parts = _counts_call([rev_d, acc_d, tit_d, mgr_d, dep_d],
                             [nu, nr, nu, nu, nu])
    inv_rev, inv_acc, inv_t, inv_m, inv_d = (
        1.0 / jnp.maximum(cp[0, :, 0] + cp[1, :, 0], 1.0) for cp in cnt_parts)

    agg_t = _agg_call(p['Title_emb'].reshape(nt * 8, H // 8), tit_s, tit_d,
                      nu, 8)
    agg_m = _agg_call(p['Manager_emb'].reshape(nm * 8, H // 8), mgr_s, mgr_d,
                      nu, 8)
    agg_d = _agg_call(p['Department_emb'].reshape(nd_ * 8, H // 8),
                      dep_s, dep_d, nu, 8)

    wr_sum1 = (p['l1_rev']['W_r'] + p['l1_title']['W_r']
               + p['l1_mgr']['W_r'] + p['l1_dept']['W_r'])
    wr_sum2 = (p['l2_rev']['W_r'] + p['l2_title']['W_r']
               + p['l2_mgr']['W_r'] + p['l2_dept']['W_r'])
    b_sum1 = (p['l1_rev']['b_l'] + p['l1_title']['b_l']
              + p['l1_mgr']['b_l'] + p['l1_dept']['b_l'])
    b_sum2 = (p['l2_rev']['b_l'] + p['l2_title']['b_l']
              + p['l2_mgr']['b_l'] + p['l2_dept']['b_l'])

    xu0, pre_u1, pre_s2 = _prep_user(
        x_User, p['User_emb'], agg_t, agg_m, agg_d,
        col(inv_t), col(inv_m), col(inv_d),
        p['User_lin_W'], rowv(p['User_lin_b']),
        wr_sum1, rowv(b_sum1),
        p['l1_title']['W_l'], p['l1_mgr']['W_l'], p['l1_dept']['W_l'],
        p['l2_title']['W_l'], p['l2_mgr']['W_l'], p['l2_dept']['W_l'],
        rowv(b_sum2))
    xr0, y1 = _prep_res(x_Resource, p['Resource_emb'], p['Resource_lin_W'],
                        rowv(p['Resource_lin_b']), p['l1_rev']['W_l'])

    agg_rev1 = _agg_call(y1.reshape(nr * 8, H // 8), rev_s, rev_d, nu, 8)
    agg_acc1 = _agg_call(xu0.reshape(nu * 2, H // 2), acc_s, acc_d, nr, 2)
    u1, pre_u2 = _combine_u1(pre_u1, agg_rev1, col(inv_rev), pre_s2, wr_sum2)
    r1, y2 = _combine_r1(agg_acc1, col(inv_acc), xr0, p['l1_access']['W_l'],
                         p['l1_access']['W_r'], rowv(p['l1_access']['b_l']),
                         p['l2_rev']['W_l'])

    agg_rev2 = _agg_call(y2.reshape(nr * 8, H // 8), rev_s, rev_d, nu, 8)
    agg_acc2 = _agg_call(u1.reshape(nu * 2, H // 2), acc_s, acc_d, nr, 2)
    u2 = _combine_u2(pre_u2, agg_rev2, col(inv_rev))
    r2 = _combine_r2(agg_acc2, col(inv_acc), r1, p['l2_access']['W_l'],
                     p['l2_access']['W_r'], rowv(p['l2_access']['b_l']))

    pred = _pred_call(u2, r2, el0, el1)

    x_out = {
        'User': u2,
        'Resource': r2,
        'Title': p['Title_emb'],
        'Manager': p['Manager_emb'],
        'Department': p['Department_emb'],
    }
    return pred, x_out
